# baseline (speedup 1.0000x reference)
import functools

import jax
import jax.numpy as jnp
from jax import lax
from jax.experimental import pallas as pl
from jax.experimental.pallas import tpu as pltpu
from jax.experimental.pallas import tpu_sc as plsc

NUM_BINS = 15
EPS = 1e-9
ROWS = 16384
COLS = 1000

_NC = 2
_NS = 16
_NW = _NC * _NS
_L = 16
_CR = 16
_PW = 2 * COLS
_NV = _PW // _L
_BV = COLS // _L


def _tc_body(x_ref, u_ref, cal_ref, o_ref):
    x = x_ref[...]
    m = jnp.max(x, axis=1, keepdims=True)
    e = jnp.exp(x - m)
    s = jnp.sum(e, axis=1, keepdims=True)
    p = jnp.clip(e / s, EPS, 1.0 - EPS)
    acc = jnp.full(x.shape, cal_ref[0], dtype=jnp.float32)
    for j in range(NUM_BINS - 1):
        acc = acc + jnp.where(p >= u_ref[j], cal_ref[j + 1], 0.0)
    o_ref[...] = acc


def _tc_calibrate(logits, u, cal, block_rows):
    grid = logits.shape[0] // block_rows
    return pl.pallas_call(
        _tc_body,
        grid=(grid,),
        in_specs=[
            pl.BlockSpec((block_rows, COLS), lambda i: (i, 0)),
            pl.BlockSpec(memory_space=pltpu.SMEM),
            pl.BlockSpec(memory_space=pltpu.SMEM),
        ],
        out_specs=pl.BlockSpec((block_rows, COLS), lambda i: (i, 0)),
        out_shape=jax.ShapeDtypeStruct(logits.shape, jnp.float32),
    )(logits, u, cal)


def _sc_calibrate(logits_flat, u16, t16, n_rows, interpret=False):
    rpw = n_rows // _NW
    nk = rpw // _CR
    cw = _CR * COLS

    def body(x_hbm, u_hbm, t_hbm, o_hbm, st, ob, uv, tv):
        wid = lax.axis_index("s") * _NC + lax.axis_index("c")
        pltpu.sync_copy(u_hbm, uv)
        pltpu.sync_copy(t_hbm, tv)
        lane = lax.iota(jnp.int32, 16)
        lo8 = lane < 8
        uvec = uv[...]
        tvec = tv[...]

        vt = [jnp.full((_L,), tvec[j], jnp.float32) for j in range(NUM_BINS)]
        ninf = jnp.full((_L,), -jnp.inf, jnp.float32)
        zero = jnp.zeros((_L,), jnp.float32)

        def row_thresholds(s_scalar):
            out = []
            for j in range(NUM_BINS - 1):
                uj = uvec[j]
                w = jnp.where(uj <= jnp.float32(EPS), -jnp.inf,
                              jnp.where(uj > jnp.float32(1.0 - EPS), jnp.inf,
                                        uj * s_scalar))
                out.append(jnp.full((_L,), w, jnp.float32))
            return out

        def chain(e, vth):
            acc = vt[0]
            for j in range(NUM_BINS - 1):
                acc = jnp.where(e >= vth[j], vt[j + 1], acc)
            return acc

        def pair(pb, carry0):
            base = pb * _PW

            def maxloop(off0):
                @plsc.parallel_loop(0, _BV, 1, unroll=31, carry=ninf)
                def step(ci, acc):
                    return jnp.maximum(acc, st[pl.ds(off0 + ci * _L, _L)])
                return step
            accA = maxloop(base)
            accB = maxloop(base + (_BV + 1) * _L)
            vb = st[pl.ds(base + _BV * _L, _L)]
            mA = jnp.max(jnp.maximum(accA, jnp.where(lo8, vb, -jnp.inf)))
            mB = jnp.max(jnp.maximum(accB, jnp.where(lo8, -jnp.inf, vb)))
            mAv = jnp.full((_L,), mA, jnp.float32)
            mBv = jnp.full((_L,), mB, jnp.float32)

            def exploop(off0, mv):
                @plsc.parallel_loop(0, _BV, 1, unroll=31, carry=zero)
                def step(ci, acc):
                    off = off0 + ci * _L
                    e = jnp.exp(st[pl.ds(off, _L)] - mv)
                    st[pl.ds(off, _L)] = e
                    return acc + e
                return step
            sumA = exploop(base, mAv)
            sumB = exploop(base + (_BV + 1) * _L, mBv)
            eb = jnp.exp(vb - jnp.where(lo8, mAv, mBv))
            st[pl.ds(base + _BV * _L, _L)] = eb
            sA = jnp.sum(sumA) + jnp.sum(jnp.where(lo8, eb, 0.0))
            sB = jnp.sum(sumB) + jnp.sum(jnp.where(lo8, 0.0, eb))

            vthA = row_thresholds(sA)
            vthB = row_thresholds(sB)

            def calloop(off0, vth):
                @plsc.parallel_loop(0, _BV, 1, unroll=31)
                def step(ci):
                    off = off0 + ci * _L
                    ob[pl.ds(off, _L)] = chain(st[pl.ds(off, _L)], vth)
            calloop(base, vthA)
            calloop(base + (_BV + 1) * _L, vthB)
            vthP = [jnp.where(lo8, a, b) for a, b in zip(vthA, vthB)]
            ob[pl.ds(base + _BV * _L, _L)] = chain(eb, vthP)
            return carry0

        def chunk(k, carry0):
            w0 = (wid * nk + k) * cw
            pltpu.sync_copy(x_hbm.at[pl.ds(w0, cw)], st)
            lax.fori_loop(0, _CR // 2, pair, 0)
            pltpu.sync_copy(ob, o_hbm.at[pl.ds(w0, cw)])
            return carry0

        lax.fori_loop(0, nk, chunk, 0)

    return pl.kernel(
        body,
        out_type=jax.ShapeDtypeStruct((n_rows * COLS,), jnp.float32),
        mesh=plsc.VectorSubcoreMesh(core_axis_name="c", subcore_axis_name="s"),
        compiler_params=pltpu.CompilerParams(needs_layout_passes=False),
        scratch_types=[
            pltpu.VMEM((cw,), jnp.float32),
            pltpu.VMEM((cw,), jnp.float32),
            pltpu.VMEM((16,), jnp.float32),
            pltpu.VMEM((16,), jnp.float32),
        ],
        interpret=interpret,
    )(logits_flat, u16, t16)


@jax.jit
def kernel(logits, bin_boundaries, bin_reprs):
    u = jax.nn.sigmoid(bin_boundaries)
    t = jax.nn.sigmoid(bin_reprs)
    u16 = jnp.pad(u, (0, 2))
    t16 = jnp.pad(t, (0, 1))
    out_flat = _sc_calibrate(logits.reshape(-1), u16, t16, ROWS)
    return out_flat.reshape(logits.shape)


# speedup vs baseline: 3.3152x; 3.3152x over previous
import functools

import jax
import jax.numpy as jnp
from jax import lax
from jax.experimental import pallas as pl
from jax.experimental.pallas import tpu as pltpu
from jax.experimental.pallas import tpu_sc as plsc

NUM_BINS = 15
EPS = 1e-9
ROWS = 16384
COLS = 1000

_NC = 2
_NS = 16
_NW = _NC * _NS
_L = 16
_CR = 16
_PW = 2 * COLS
_NV = _PW // _L
_BV = COLS // _L


def _tc_body(x_ref, u_ref, cal_ref, o_ref):
    x = x_ref[...]
    m = jnp.max(x, axis=1, keepdims=True)
    e = jnp.exp(x - m)
    s = jnp.sum(e, axis=1, keepdims=True)
    p = jnp.clip(e / s, EPS, 1.0 - EPS)
    acc = jnp.full(x.shape, cal_ref[0], dtype=jnp.float32)
    for j in range(NUM_BINS - 1):
        acc = acc + jnp.where(p >= u_ref[j], cal_ref[j + 1], 0.0)
    o_ref[...] = acc


def _tc_calibrate(logits, u, cal, block_rows):
    grid = logits.shape[0] // block_rows
    return pl.pallas_call(
        _tc_body,
        grid=(grid,),
        in_specs=[
            pl.BlockSpec((block_rows, COLS), lambda i: (i, 0)),
            pl.BlockSpec(memory_space=pltpu.SMEM),
            pl.BlockSpec(memory_space=pltpu.SMEM),
        ],
        out_specs=pl.BlockSpec((block_rows, COLS), lambda i: (i, 0)),
        out_shape=jax.ShapeDtypeStruct(logits.shape, jnp.float32),
    )(logits, u, cal)


def _sc_calibrate(logits_flat, u16, t16, n_rows, interpret=False):
    rpw = n_rows // _NW
    nk = rpw // _CR
    cw = _CR * COLS

    def body(x_hbm, u_hbm, t_hbm, o_hbm, st, ob, uv, tv):
        wid = lax.axis_index("s") * _NC + lax.axis_index("c")
        pltpu.sync_copy(u_hbm, uv)
        pltpu.sync_copy(t_hbm, tv)
        lane = lax.iota(jnp.int32, 16)
        lo8 = lane < 8
        uvec = uv[...]
        tvec = tv[...]

        vt = [jnp.full((_L,), tvec[j], jnp.float32) for j in range(NUM_BINS)]
        ninf = jnp.full((_L,), -jnp.inf, jnp.float32)
        zero = jnp.zeros((_L,), jnp.float32)

        def row_thresholds(s_scalar):
            out = []
            for j in range(NUM_BINS - 1):
                uj = uvec[j]
                w = jnp.where(uj <= jnp.float32(EPS), -jnp.inf,
                              jnp.where(uj > jnp.float32(1.0 - EPS), jnp.inf,
                                        uj * s_scalar))
                out.append(jnp.full((_L,), w, jnp.float32))
            return out

        def chain(e, vth):
            acc = vt[0]
            for j in range(NUM_BINS - 1):
                acc = jnp.where(e >= vth[j], vt[j + 1], acc)
            return acc

        def pair(pb, carry0):
            base = pb * _PW

            def maxloop(off0):
                @plsc.parallel_loop(0, _BV, 1, unroll=8, carry=ninf)
                def step(ci, acc):
                    return jnp.maximum(acc, st[pl.ds(off0 + ci * _L, _L)])
                return step
            with jax.named_scope("passA"):
                accA = maxloop(base)
                accB = maxloop(base + (_BV + 1) * _L)
            vb = st[pl.ds(base + _BV * _L, _L)]
            mA = jnp.max(jnp.maximum(accA, jnp.where(lo8, vb, -jnp.inf)))
            mB = jnp.max(jnp.maximum(accB, jnp.where(lo8, -jnp.inf, vb)))
            mAv = jnp.full((_L,), mA, jnp.float32)
            mBv = jnp.full((_L,), mB, jnp.float32)

            def exploop(off0, mv):
                @plsc.parallel_loop(0, _BV, 1, unroll=8, carry=zero)
                def step(ci, acc):
                    off = off0 + ci * _L
                    e = jnp.exp(st[pl.ds(off, _L)] - mv)
                    st[pl.ds(off, _L)] = e
                    return acc + e
                return step
            with jax.named_scope("passB"):
                sumA = exploop(base, mAv)
                sumB = exploop(base + (_BV + 1) * _L, mBv)
            eb = jnp.exp(vb - jnp.where(lo8, mAv, mBv))
            st[pl.ds(base + _BV * _L, _L)] = eb
            sA = jnp.sum(sumA) + jnp.sum(jnp.where(lo8, eb, 0.0))
            sB = jnp.sum(sumB) + jnp.sum(jnp.where(lo8, 0.0, eb))

            vthA = row_thresholds(sA)
            vthB = row_thresholds(sB)

            def calloop(off0, vth):
                @plsc.parallel_loop(0, _BV, 1, unroll=8)
                def step(ci):
                    off = off0 + ci * _L
                    ob[pl.ds(off, _L)] = chain(st[pl.ds(off, _L)], vth)
            with jax.named_scope("passC"):
                calloop(base, vthA)
                calloop(base + (_BV + 1) * _L, vthB)
            vthP = [jnp.where(lo8, a, b) for a, b in zip(vthA, vthB)]
            ob[pl.ds(base + _BV * _L, _L)] = chain(eb, vthP)
            return carry0

        def chunk(k, carry0):
            w0 = (wid * nk + k) * cw
            pltpu.sync_copy(x_hbm.at[pl.ds(w0, cw)], st)
            lax.fori_loop(0, _CR // 2, pair, 0)
            pltpu.sync_copy(ob, o_hbm.at[pl.ds(w0, cw)])
            return carry0

        lax.fori_loop(0, nk, chunk, 0)

    return pl.kernel(
        body,
        out_type=jax.ShapeDtypeStruct((n_rows * COLS,), jnp.float32),
        mesh=plsc.VectorSubcoreMesh(core_axis_name="c", subcore_axis_name="s"),
        compiler_params=pltpu.CompilerParams(needs_layout_passes=False),
        scratch_types=[
            pltpu.VMEM((cw,), jnp.float32),
            pltpu.VMEM((cw,), jnp.float32),
            pltpu.VMEM((16,), jnp.float32),
            pltpu.VMEM((16,), jnp.float32),
        ],
        interpret=interpret,
    )(logits_flat, u16, t16)


@jax.jit
def kernel(logits, bin_boundaries, bin_reprs):
    u = jax.nn.sigmoid(bin_boundaries)
    t = jax.nn.sigmoid(bin_reprs)
    u16 = jnp.pad(u, (0, 2))
    t16 = jnp.pad(t, (0, 1))
    out_flat = _sc_calibrate(logits.reshape(-1), u16, t16, ROWS)
    return out_flat.reshape(logits.shape)


# speedup vs baseline: 6.3679x; 1.9208x over previous
import functools

import jax
import jax.numpy as jnp
from jax import lax
from jax.experimental import pallas as pl
from jax.experimental.pallas import tpu as pltpu
from jax.experimental.pallas import tpu_sc as plsc

NUM_BINS = 15
EPS = 1e-9
ROWS = 16384
COLS = 1000

_NC = 2
_NS = 16
_NW = _NC * _NS
_L = 16
_CR = 16
_PW = 2 * COLS
_NV = _PW // _L
_BV = COLS // _L


def _tc_body(x_ref, u_ref, cal_ref, o_ref):
    x = x_ref[...]
    m = jnp.max(x, axis=1, keepdims=True)
    e = jnp.exp(x - m)
    s = jnp.sum(e, axis=1, keepdims=True)
    p = jnp.clip(e / s, EPS, 1.0 - EPS)
    acc = jnp.full(x.shape, cal_ref[0], dtype=jnp.float32)
    for j in range(NUM_BINS - 1):
        acc = acc + jnp.where(p >= u_ref[j], cal_ref[j + 1], 0.0)
    o_ref[...] = acc


def _tc_calibrate(logits, u, cal, block_rows, blk0=0, n_blocks=None):
    if n_blocks is None:
        n_blocks = logits.shape[0] // block_rows - blk0
    return pl.pallas_call(
        _tc_body,
        grid=(n_blocks,),
        in_specs=[
            pl.BlockSpec((block_rows, COLS), lambda i: (i + blk0, 0)),
            pl.BlockSpec(memory_space=pltpu.SMEM),
            pl.BlockSpec(memory_space=pltpu.SMEM),
        ],
        out_specs=pl.BlockSpec((block_rows, COLS), lambda i: (i + blk0, 0)),
        out_shape=jax.ShapeDtypeStruct(logits.shape, jnp.float32),
    )(logits, u, cal)


def _sc_calibrate(logits_flat, u16, t16, n_rows, interpret=False):
    rpw = n_rows // _NW
    nk = rpw // _CR
    cw = _CR * COLS

    def body(x_hbm, u_hbm, t_hbm, o_hbm, st, ob, uv, tv):
        wid = lax.axis_index("s") * _NC + lax.axis_index("c")
        pltpu.sync_copy(u_hbm, uv)
        pltpu.sync_copy(t_hbm, tv)
        lane = lax.iota(jnp.int32, 16)
        lo8 = lane < 8
        uvec = uv[...]
        tvec = tv[...]

        vt = [jnp.full((_L,), tvec[j], jnp.float32) for j in range(NUM_BINS)]
        ninf = jnp.full((_L,), -jnp.inf, jnp.float32)
        zero = jnp.zeros((_L,), jnp.float32)

        def row_thresholds(s_scalar):
            out = []
            for j in range(NUM_BINS - 1):
                uj = uvec[j]
                w = jnp.where(uj <= jnp.float32(EPS), -jnp.inf,
                              jnp.where(uj > jnp.float32(1.0 - EPS), jnp.inf,
                                        uj * s_scalar))
                out.append(jnp.full((_L,), w, jnp.float32))
            return out

        def chain(e, vth):
            acc = vt[0]
            for j in range(NUM_BINS - 1):
                acc = jnp.where(e >= vth[j], vt[j + 1], acc)
            return acc

        def pair(pb, carry0):
            base = pb * _PW

            def maxloop(off0):
                @plsc.parallel_loop(0, _BV, 1, unroll=8, carry=ninf)
                def step(ci, acc):
                    return jnp.maximum(acc, st[pl.ds(off0 + ci * _L, _L)])
                return step
            with jax.named_scope("passA"):
                accA = maxloop(base)
                accB = maxloop(base + (_BV + 1) * _L)
            vb = st[pl.ds(base + _BV * _L, _L)]
            mA = jnp.max(jnp.maximum(accA, jnp.where(lo8, vb, -jnp.inf)))
            mB = jnp.max(jnp.maximum(accB, jnp.where(lo8, -jnp.inf, vb)))
            mAv = jnp.full((_L,), mA, jnp.float32)
            mBv = jnp.full((_L,), mB, jnp.float32)

            def exploop(off0, mv):
                @plsc.parallel_loop(0, _BV, 1, unroll=8, carry=zero)
                def step(ci, acc):
                    off = off0 + ci * _L
                    e = jnp.exp(st[pl.ds(off, _L)] - mv)
                    st[pl.ds(off, _L)] = e
                    return acc + e
                return step
            with jax.named_scope("passB"):
                sumA = exploop(base, mAv)
                sumB = exploop(base + (_BV + 1) * _L, mBv)
            eb = jnp.exp(vb - jnp.where(lo8, mAv, mBv))
            st[pl.ds(base + _BV * _L, _L)] = eb
            sA = jnp.sum(sumA) + jnp.sum(jnp.where(lo8, eb, 0.0))
            sB = jnp.sum(sumB) + jnp.sum(jnp.where(lo8, 0.0, eb))

            vthA = row_thresholds(sA)
            vthB = row_thresholds(sB)

            def calloop(off0, vth):
                @plsc.parallel_loop(0, _BV, 1, unroll=8)
                def step(ci):
                    off = off0 + ci * _L
                    ob[pl.ds(off, _L)] = chain(st[pl.ds(off, _L)], vth)
            with jax.named_scope("passC"):
                calloop(base, vthA)
                calloop(base + (_BV + 1) * _L, vthB)
            vthP = [jnp.where(lo8, a, b) for a, b in zip(vthA, vthB)]
            ob[pl.ds(base + _BV * _L, _L)] = chain(eb, vthP)
            return carry0

        def chunk(k, carry0):
            w0 = (wid * nk + k) * cw
            pltpu.sync_copy(x_hbm.at[pl.ds(w0, cw)], st)
            lax.fori_loop(0, _CR // 2, pair, 0)
            pltpu.sync_copy(ob, o_hbm.at[pl.ds(w0, cw)])
            return carry0

        lax.fori_loop(0, nk, chunk, 0)

    return pl.kernel(
        body,
        out_type=jax.ShapeDtypeStruct((n_rows * COLS,), jnp.float32),
        mesh=plsc.VectorSubcoreMesh(core_axis_name="c", subcore_axis_name="s"),
        compiler_params=pltpu.CompilerParams(needs_layout_passes=False),
        scratch_types=[
            pltpu.VMEM((cw,), jnp.float32),
            pltpu.VMEM((cw,), jnp.float32),
            pltpu.VMEM((16,), jnp.float32),
            pltpu.VMEM((16,), jnp.float32),
        ],
        interpret=interpret,
    )(logits_flat, u16, t16)


_SC_ROWS = 5120
_TC_BLOCK = 256


@jax.jit
def kernel(logits, bin_boundaries, bin_reprs):
    u = jax.nn.sigmoid(bin_boundaries)
    t = jax.nn.sigmoid(bin_reprs)
    cal = jnp.concatenate([t[:1], jnp.diff(t)])
    u16 = jnp.pad(u, (0, 2))
    t16 = jnp.pad(t, (0, 1))
    sc_flat = _sc_calibrate(logits[:_SC_ROWS].reshape(-1), u16, t16, _SC_ROWS)
    tc_full = _tc_calibrate(logits, u, cal, _TC_BLOCK,
                            blk0=_SC_ROWS // _TC_BLOCK)
    return lax.dynamic_update_slice(
        tc_full, sc_flat.reshape(_SC_ROWS, COLS), (0, 0))


# speedup vs baseline: 6.6699x; 1.0474x over previous
import functools

import jax
import jax.numpy as jnp
from jax import lax
from jax.experimental import pallas as pl
from jax.experimental.pallas import tpu as pltpu
from jax.experimental.pallas import tpu_sc as plsc

NUM_BINS = 15
EPS = 1e-9
ROWS = 16384
COLS = 1000

_NC = 2
_NS = 16
_NW = _NC * _NS
_L = 16
_CR = 16
_PW = 2 * COLS
_NV = _PW // _L
_BV = COLS // _L


def _tc_body(x_ref, u_ref, cal_ref, o_ref):
    x = x_ref[...]
    m = jnp.max(x, axis=1, keepdims=True)
    e = jnp.exp(x - m)
    s = jnp.sum(e, axis=1, keepdims=True)
    p = jnp.clip(e / s, EPS, 1.0 - EPS)
    acc = jnp.full(x.shape, cal_ref[0], dtype=jnp.float32)
    for j in range(NUM_BINS - 1):
        acc = acc + jnp.where(p >= u_ref[j], cal_ref[j + 1], 0.0)
    o_ref[...] = acc


def _tc_calibrate(logits, u, cal, block_rows, blk0=0, n_blocks=None):
    if n_blocks is None:
        n_blocks = logits.shape[0] // block_rows - blk0
    return pl.pallas_call(
        _tc_body,
        grid=(n_blocks,),
        in_specs=[
            pl.BlockSpec((block_rows, COLS), lambda i: (i + blk0, 0)),
            pl.BlockSpec(memory_space=pltpu.SMEM),
            pl.BlockSpec(memory_space=pltpu.SMEM),
        ],
        out_specs=pl.BlockSpec((block_rows, COLS), lambda i: (i + blk0, 0)),
        out_shape=jax.ShapeDtypeStruct(logits.shape, jnp.float32),
    )(logits, u, cal)


def _sc_calibrate(logits_flat, u16, t16, n_rows, interpret=False):
    rpw = n_rows // _NW
    nk = rpw // _CR
    cw = _CR * COLS

    def body(x_hbm, u_hbm, t_hbm, o_hbm, st, ob, uv, tv):
        wid = lax.axis_index("s") * _NC + lax.axis_index("c")
        pltpu.sync_copy(u_hbm, uv)
        pltpu.sync_copy(t_hbm, tv)
        lane = lax.iota(jnp.int32, 16)
        lo8 = lane < 8
        uvec = uv[...]
        tvec = tv[...]

        vt = [jnp.full((_L,), tvec[j], jnp.float32) for j in range(NUM_BINS)]
        ninf = jnp.full((_L,), -jnp.inf, jnp.float32)
        zero = jnp.zeros((_L,), jnp.float32)

        def row_thresholds(s_scalar):
            out = []
            for j in range(NUM_BINS - 1):
                uj = uvec[j]
                w = jnp.where(uj <= jnp.float32(EPS), -jnp.inf,
                              jnp.where(uj > jnp.float32(1.0 - EPS), jnp.inf,
                                        uj * s_scalar))
                out.append(jnp.full((_L,), w, jnp.float32))
            return out

        def chain(e, vth):
            acc = vt[0]
            for j in range(NUM_BINS - 1):
                acc = jnp.where(e >= vth[j], vt[j + 1], acc)
            return acc

        def pair(pb, carry0):
            base = pb * _PW

            def maxloop(off0):
                @plsc.parallel_loop(0, _BV, 1, unroll=8, carry=ninf)
                def step(ci, acc):
                    return jnp.maximum(acc, st[pl.ds(off0 + ci * _L, _L)])
                return step
            with jax.named_scope("passA"):
                accA = maxloop(base)
                accB = maxloop(base + (_BV + 1) * _L)
            vb = st[pl.ds(base + _BV * _L, _L)]
            mA = jnp.max(jnp.maximum(accA, jnp.where(lo8, vb, -jnp.inf)))
            mB = jnp.max(jnp.maximum(accB, jnp.where(lo8, -jnp.inf, vb)))
            mAv = jnp.full((_L,), mA, jnp.float32)
            mBv = jnp.full((_L,), mB, jnp.float32)

            def exploop(off0, mv):
                @plsc.parallel_loop(0, _BV, 1, unroll=8, carry=zero)
                def step(ci, acc):
                    off = off0 + ci * _L
                    e = jnp.exp(st[pl.ds(off, _L)] - mv)
                    st[pl.ds(off, _L)] = e
                    return acc + e
                return step
            with jax.named_scope("passB"):
                sumA = exploop(base, mAv)
                sumB = exploop(base + (_BV + 1) * _L, mBv)
            eb = jnp.exp(vb - jnp.where(lo8, mAv, mBv))
            st[pl.ds(base + _BV * _L, _L)] = eb
            sA = jnp.sum(sumA) + jnp.sum(jnp.where(lo8, eb, 0.0))
            sB = jnp.sum(sumB) + jnp.sum(jnp.where(lo8, 0.0, eb))

            vthA = row_thresholds(sA)
            vthB = row_thresholds(sB)

            def calloop(off0, vth):
                @plsc.parallel_loop(0, _BV, 1, unroll=8)
                def step(ci):
                    off = off0 + ci * _L
                    ob[pl.ds(off, _L)] = chain(st[pl.ds(off, _L)], vth)
            with jax.named_scope("passC"):
                calloop(base, vthA)
                calloop(base + (_BV + 1) * _L, vthB)
            vthP = [jnp.where(lo8, a, b) for a, b in zip(vthA, vthB)]
            ob[pl.ds(base + _BV * _L, _L)] = chain(eb, vthP)
            return carry0

        def chunk(k, carry0):
            w0 = (wid * nk + k) * cw
            pltpu.sync_copy(x_hbm.at[pl.ds(w0, cw)], st)
            lax.fori_loop(0, _CR // 2, pair, 0)
            pltpu.sync_copy(ob, o_hbm.at[pl.ds(w0, cw)])
            return carry0

        lax.fori_loop(0, nk, chunk, 0)

    return pl.kernel(
        body,
        out_type=jax.ShapeDtypeStruct((n_rows * COLS,), jnp.float32),
        mesh=plsc.VectorSubcoreMesh(core_axis_name="c", subcore_axis_name="s"),
        compiler_params=pltpu.CompilerParams(needs_layout_passes=False),
        scratch_types=[
            pltpu.VMEM((cw,), jnp.float32),
            pltpu.VMEM((cw,), jnp.float32),
            pltpu.VMEM((16,), jnp.float32),
            pltpu.VMEM((16,), jnp.float32),
        ],
        interpret=interpret,
    )(logits_flat, u16, t16)


_SC_ROWS = 2048
_TC_BLOCK = 256


@jax.jit
def kernel(logits, bin_boundaries, bin_reprs):
    u = jax.nn.sigmoid(bin_boundaries)
    t = jax.nn.sigmoid(bin_reprs)
    cal = jnp.concatenate([t[:1], jnp.diff(t)])
    u16 = jnp.pad(u, (0, 2))
    t16 = jnp.pad(t, (0, 1))
    sc_flat = _sc_calibrate(logits[:_SC_ROWS].reshape(-1), u16, t16, _SC_ROWS)
    tc_full = _tc_calibrate(logits, u, cal, _TC_BLOCK,
                            blk0=_SC_ROWS // _TC_BLOCK)
    return lax.dynamic_update_slice(
        tc_full, sc_flat.reshape(_SC_ROWS, COLS), (0, 0))
